# baseline (speedup 1.0000x reference)
import jax
import jax.numpy as jnp
from jax import lax
from jax.experimental import pallas as pl
from jax.experimental.pallas import tpu as pltpu
from jax.experimental.pallas import tpu_sc as plsc

NUM_EMB = 1000000
DIM = 64
NSEQ = 16384
SEQ = 50
NW = 32
SEQ_PER_W = NSEQ // NW
NCHUNK = SEQ_PER_W
NBUF = 8


def _body(idx_hbm, table_hbm, out_hbm, idx_v, *rest):
    bufs = rest[:NBUF]
    gsems = rest[NBUF:2 * NBUF]
    ssems = rest[2 * NBUF:3 * NBUF]
    nc = 2
    wid = lax.axis_index("s") * nc + lax.axis_index("c")
    seq0 = wid * SEQ_PER_W
    pltpu.sync_copy(idx_hbm.at[pl.ds(seq0, SEQ_PER_W), :], idx_v)

    def out_slice(j):
        return out_hbm.at[seq0 + j]

    def gather(j, b):
        return pltpu.make_async_copy(
            table_hbm.at[idx_v.at[j]], bufs[b], gsems[b])

    def store(j, b):
        return pltpu.make_async_copy(bufs[b], out_slice(j), ssems[b])

    for b in range(NBUF):
        gather(b, b).start()

    @pl.loop(0, NCHUNK - NBUF, step=NBUF)
    def _round(g):
        for b in range(NBUF):
            j = g + b
            gather(j, b).wait()
            store(j, b).start()
        for b in range(NBUF):
            j = g + b
            store(j, b).wait()
            gather(j + NBUF, b).start()

    g_last = NCHUNK - NBUF
    for b in range(NBUF):
        j = g_last + b
        gather(j, b).wait()
        store(j, b).start()
    for b in range(NBUF):
        store(g_last + b, b).wait()


@jax.jit
def _gather_sc(idx, table):
    mesh = plsc.VectorSubcoreMesh(core_axis_name="c", subcore_axis_name="s")
    return pl.kernel(
        _body,
        out_type=jax.ShapeDtypeStruct((NSEQ, SEQ, DIM), jnp.float32),
        mesh=mesh,
        scratch_types=(
            [pltpu.VMEM((SEQ_PER_W, SEQ), jnp.int32)]
            + [pltpu.VMEM((SEQ, DIM), jnp.float32) for _ in range(NBUF)]
            + [pltpu.SemaphoreType.DMA for _ in range(2 * NBUF)]
        ),
        compiler_params=pltpu.CompilerParams(use_tc_tiling_on_sc=False),
    )(idx, table)


def kernel(indices, table):
    return _gather_sc(indices, table)
